# baseline (speedup 1.0000x reference)
import functools

import jax
import jax.numpy as jnp
from jax import lax
from jax.experimental import pallas as pl
from jax.experimental.pallas import tpu as pltpu
from jax.experimental.pallas import tpu_sc as plsc

N_NODES = 10000
N_EDGES = 320000
D_IN = 128
D_OUT = 128
D_AUG = 144

NC = 2
NS = 16
NW = NC * NS
EDGES_PER_TILE = N_EDGES // NW
CHUNK = 80
NCHUNK = EDGES_PER_TILE // CHUNK
N_PAD = 10240
ROWS_PER_TILE = N_PAD // NS
ZROWS = 128


def _sc_aggregate(x_aug, src, dst):
  mesh = plsc.VectorSubcoreMesh(core_axis_name="c", subcore_axis_name="s")

  @functools.partial(
      pl.kernel,
      mesh=mesh,
      compiler_params=pltpu.CompilerParams(use_tc_tiling_on_sc=False),
      out_type=jax.ShapeDtypeStruct((NC, N_PAD, D_AUG), jnp.float32),
      scratch_types=[
          pltpu.VMEM((2, CHUNK), jnp.int32),
          pltpu.VMEM((2, CHUNK), jnp.int32),
          pltpu.VMEM((2, CHUNK), jnp.int32),
          pltpu.VMEM((2, CHUNK), jnp.int32),
          pltpu.VMEM((CHUNK, D_AUG), jnp.float32),
          pltpu.VMEM((CHUNK, D_AUG), jnp.float32),
          pltpu.VMEM_SHARED((N_PAD, D_AUG), jnp.float32),
          pltpu.SemaphoreType.DMA,
          pltpu.SemaphoreType.DMA,
          pltpu.SemaphoreType.DMA,
          pltpu.SemaphoreType.DMA,
          pltpu.SemaphoreType.DMA,
          pltpu.SemaphoreType.DMA,
          pltpu.SemaphoreType.DMA,
          pltpu.SemaphoreType.DMA,
      ],
  )
  def body(xaug_hbm, eidx_hbm, out_hbm, e0, e1, e2, e3, rows0, rows1, acc,
           is0, is1, is2, is3, gs0, gs1, ss0, ss1):
    c = lax.axis_index("c")
    s = lax.axis_index("s")
    wid = s * NC + c

    ebufs = (e0, e1, e2, e3)
    isems = (is0, is1, is2, is3)
    rbufs = (rows0, rows1)
    gsems = (gs0, gs1)
    ssems = (ss0, ss1)

    def zero_row(r, carry):
      for k in range(D_AUG // 16):
        rows0[r, pl.ds(k * 16, 16)] = jnp.zeros((16,), jnp.float32)
      return carry

    lax.fori_loop(0, CHUNK, zero_row, 0)

    def zero_acc(i, carry):
      pltpu.sync_copy(
          rows0, acc.at[pl.ds(s * ROWS_PER_TILE + i * CHUNK, CHUNK)])
      return carry

    lax.fori_loop(0, ROWS_PER_TILE // CHUNK, zero_acc, 0)
    plsc.subcore_barrier()

    def start_idx(j, ib):
      pltpu.async_copy(eidx_hbm.at[wid, j], ebufs[ib], isems[ib])

    def wait_idx(ib):
      pltpu.make_async_copy(eidx_hbm.at[0, 0], ebufs[ib], isems[ib]).wait()

    def start_gather(ib, rb):
      pltpu.async_copy(xaug_hbm.at[ebufs[ib].at[0]], rbufs[rb], gsems[rb])

    def wait_gather(rb):
      pltpu.make_async_copy(xaug_hbm.at[pl.ds(0, CHUNK)], rbufs[rb],
                            gsems[rb]).wait()

    def start_scatter(ib, rb):
      pltpu.async_copy(rbufs[rb], acc.at[ebufs[ib].at[1]], ssems[rb],
                       add=True)

    def wait_scatter(rb):
      pltpu.make_async_copy(rbufs[rb], acc.at[pl.ds(0, CHUNK)],
                            ssems[rb]).wait()

    pltpu.sync_copy(eidx_hbm.at[wid, 0], e0)
    start_gather(0, 0)
    start_idx(1, 1)

    def quad(p, carry):
      j0 = 4 * p
      for b in range(4):
        j = j0 + b
        ib = b
        rb = b % 2

        @pl.when(j < NCHUNK)
        def _():
          @pl.when(j + 1 < NCHUNK)
          def _():
            wait_idx((ib + 1) % 4)

            @pl.when(j >= 1)
            def _():
              wait_scatter(1 - rb)

            start_gather((ib + 1) % 4, 1 - rb)

          @pl.when(j + 2 < NCHUNK)
          def _():
            start_idx(j + 2, (ib + 2) % 4)

          wait_gather(rb)
          start_scatter(ib, rb)

      return carry

    lax.fori_loop(0, (NCHUNK + 3) // 4, quad, 0)
    wait_scatter(NCHUNK % 2)
    wait_scatter((NCHUNK + 1) % 2)
    plsc.subcore_barrier()

    def out_step(i, carry):
      r0 = s * ROWS_PER_TILE + i * CHUNK
      pltpu.sync_copy(acc.at[pl.ds(r0, CHUNK)], rows0)
      pltpu.sync_copy(rows0, out_hbm.at[c, pl.ds(r0, CHUNK)])
      return carry

    lax.fori_loop(0, ROWS_PER_TILE // CHUNK, out_step, 0)

  eidx = jnp.stack(
      [src.reshape(NW, NCHUNK, CHUNK), dst.reshape(NW, NCHUNK, CHUNK)],
      axis=2)
  return body(x_aug, eidx)


BLK = 1000


def _combine(parts, x, w_comb, w_r_t):
  def body(p_ref, x_ref, wc_ref, wr_ref, o_ref):
    acc = p_ref[0] + p_ref[1]
    o_ref[...] = jnp.dot(
        acc, wc_ref[...], preferred_element_type=jnp.float32,
        precision=lax.Precision.HIGHEST) + jnp.dot(
            x_ref[...], wr_ref[...], preferred_element_type=jnp.float32,
            precision=lax.Precision.HIGHEST)

  return pl.pallas_call(
      body,
      grid=(N_NODES // BLK,),
      in_specs=[
          pl.BlockSpec((NC, BLK, D_AUG), lambda i: (0, i, 0)),
          pl.BlockSpec((BLK, D_IN), lambda i: (i, 0)),
          pl.BlockSpec((D_AUG, D_OUT), lambda i: (0, 0)),
          pl.BlockSpec((D_IN, D_OUT), lambda i: (0, 0)),
      ],
      out_specs=pl.BlockSpec((BLK, D_OUT), lambda i: (i, 0)),
      out_shape=jax.ShapeDtypeStruct((N_NODES, D_OUT), jnp.float32),
  )(parts, x, w_comb, w_r_t)


def kernel(x, edge_index, W_l, b_l, W_r):
  src = edge_index[0].astype(jnp.int32)
  dst = edge_index[1].astype(jnp.int32)
  x_aug = jnp.concatenate(
      [x, jnp.ones((N_NODES, 1), jnp.float32),
       jnp.zeros((N_NODES, D_AUG - D_IN - 1), jnp.float32)], axis=1)
  parts = _sc_aggregate(x_aug, src, dst)
  w_comb = jnp.concatenate(
      [W_l.T, b_l[None, :],
       jnp.zeros((D_AUG - D_IN - 1, D_OUT), jnp.float32)], axis=0)
  return _combine(parts, x, w_comb, W_r.T)


# speedup vs baseline: 1.2258x; 1.2258x over previous
import functools

import jax
import jax.numpy as jnp
from jax import lax
from jax.experimental import pallas as pl
from jax.experimental.pallas import tpu as pltpu
from jax.experimental.pallas import tpu_sc as plsc

N_NODES = 10000
N_EDGES = 320000
D_IN = 128
D_OUT = 128
D_AUG = 144

NC = 2
NS = 16
NW = NC * NS
EDGES_PER_TILE = N_EDGES // NW
CHUNK = 80
NCHUNK = EDGES_PER_TILE // CHUNK
N_PAD = 10240
ROWS_PER_TILE = N_PAD // NS
ZROWS = 128


def _sc_aggregate(x_aug, edge_idx):
  mesh = plsc.VectorSubcoreMesh(core_axis_name="c", subcore_axis_name="s")

  @functools.partial(
      pl.kernel,
      mesh=mesh,
      compiler_params=pltpu.CompilerParams(use_tc_tiling_on_sc=False),
      out_type=jax.ShapeDtypeStruct((NC, N_PAD, D_AUG), jnp.float32),
      scratch_types=[
          pltpu.VMEM((2, CHUNK), jnp.int32),
          pltpu.VMEM((2, CHUNK), jnp.int32),
          pltpu.VMEM((2, CHUNK), jnp.int32),
          pltpu.VMEM((2, CHUNK), jnp.int32),
          pltpu.VMEM((CHUNK, D_AUG), jnp.float32),
          pltpu.VMEM((CHUNK, D_AUG), jnp.float32),
          pltpu.VMEM_SHARED((N_PAD, D_AUG), jnp.float32),
          pltpu.SemaphoreType.DMA,
          pltpu.SemaphoreType.DMA,
          pltpu.SemaphoreType.DMA,
          pltpu.SemaphoreType.DMA,
          pltpu.SemaphoreType.DMA,
          pltpu.SemaphoreType.DMA,
          pltpu.SemaphoreType.DMA,
          pltpu.SemaphoreType.DMA,
      ],
  )
  def body(xaug_hbm, eidx_hbm, out_hbm, e0, e1, e2, e3, rows0, rows1, acc,
           is0, is1, is2, is3, gs0, gs1, ss0, ss1):
    c = lax.axis_index("c")
    s = lax.axis_index("s")
    wid = s * NC + c

    ebufs = (e0, e1, e2, e3)
    isems = (is0, is1, is2, is3)
    rbufs = (rows0, rows1)
    gsems = (gs0, gs1)
    ssems = (ss0, ss1)

    def zero_row(r, carry):
      for k in range(D_AUG // 16):
        rows0[r, pl.ds(k * 16, 16)] = jnp.zeros((16,), jnp.float32)
      return carry

    lax.fori_loop(0, CHUNK, zero_row, 0)

    def zero_acc(i, carry):
      pltpu.sync_copy(
          rows0, acc.at[pl.ds(s * ROWS_PER_TILE + i * CHUNK, CHUNK)])
      return carry

    lax.fori_loop(0, ROWS_PER_TILE // CHUNK, zero_acc, 0)
    plsc.subcore_barrier()

    ebase = wid * EDGES_PER_TILE

    def start_idx(j, ib):
      off = ebase + j * CHUNK
      pltpu.async_copy(eidx_hbm.at[0, pl.ds(off, CHUNK)], ebufs[ib].at[0],
                       isems[ib])
      pltpu.async_copy(eidx_hbm.at[1, pl.ds(off, CHUNK)], ebufs[ib].at[1],
                       isems[ib])

    def wait_idx(ib):
      pltpu.make_async_copy(eidx_hbm.at[pl.ds(0, 2), pl.ds(0, CHUNK)],
                            ebufs[ib], isems[ib]).wait()

    def start_gather(ib, rb):
      pltpu.async_copy(xaug_hbm.at[ebufs[ib].at[0]], rbufs[rb], gsems[rb])

    def wait_gather(rb):
      pltpu.make_async_copy(xaug_hbm.at[pl.ds(0, CHUNK)], rbufs[rb],
                            gsems[rb]).wait()

    def start_scatter(ib, rb):
      pltpu.async_copy(rbufs[rb], acc.at[ebufs[ib].at[1]], ssems[rb],
                       add=True)

    def wait_scatter(rb):
      pltpu.make_async_copy(rbufs[rb], acc.at[pl.ds(0, CHUNK)],
                            ssems[rb]).wait()

    pltpu.sync_copy(eidx_hbm.at[0, pl.ds(ebase, CHUNK)], e0.at[0])
    pltpu.sync_copy(eidx_hbm.at[1, pl.ds(ebase, CHUNK)], e0.at[1])
    start_gather(0, 0)
    start_idx(1, 1)

    def quad(p, carry):
      j0 = 4 * p
      for b in range(4):
        j = j0 + b
        ib = b
        rb = b % 2

        @pl.when(j < NCHUNK)
        def _():
          @pl.when(j + 1 < NCHUNK)
          def _():
            wait_idx((ib + 1) % 4)

            @pl.when(j >= 1)
            def _():
              wait_scatter(1 - rb)

            start_gather((ib + 1) % 4, 1 - rb)

          @pl.when(j + 2 < NCHUNK)
          def _():
            start_idx(j + 2, (ib + 2) % 4)

          wait_gather(rb)
          start_scatter(ib, rb)

      return carry

    lax.fori_loop(0, (NCHUNK + 3) // 4, quad, 0)
    wait_scatter(NCHUNK % 2)
    wait_scatter((NCHUNK + 1) % 2)
    plsc.subcore_barrier()

    def out_step(i, carry):
      r0 = s * ROWS_PER_TILE + i * CHUNK
      pltpu.sync_copy(acc.at[pl.ds(r0, CHUNK)], rows0)
      pltpu.sync_copy(rows0, out_hbm.at[c, pl.ds(r0, CHUNK)])
      return carry

    lax.fori_loop(0, ROWS_PER_TILE // CHUNK, out_step, 0)

  return body(x_aug, edge_idx)


BLK = 1000


def _combine(parts, x, w_comb, w_r_t):
  def body(p_ref, x_ref, wc_ref, wr_ref, o_ref):
    acc = p_ref[0] + p_ref[1]
    o_ref[...] = jnp.dot(
        acc, wc_ref[...], preferred_element_type=jnp.float32,
        precision=lax.Precision.HIGHEST) + jnp.dot(
            x_ref[...], wr_ref[...], preferred_element_type=jnp.float32,
            precision=lax.Precision.HIGHEST)

  return pl.pallas_call(
      body,
      grid=(N_NODES // BLK,),
      in_specs=[
          pl.BlockSpec((NC, BLK, D_AUG), lambda i: (0, i, 0)),
          pl.BlockSpec((BLK, D_IN), lambda i: (i, 0)),
          pl.BlockSpec((D_AUG, D_OUT), lambda i: (0, 0)),
          pl.BlockSpec((D_IN, D_OUT), lambda i: (0, 0)),
      ],
      out_specs=pl.BlockSpec((BLK, D_OUT), lambda i: (i, 0)),
      out_shape=jax.ShapeDtypeStruct((N_NODES, D_OUT), jnp.float32),
  )(parts, x, w_comb, w_r_t)


def kernel(x, edge_index, W_l, b_l, W_r):
  x_aug = jnp.concatenate(
      [x, jnp.ones((N_NODES, 1), jnp.float32),
       jnp.zeros((N_NODES, D_AUG - D_IN - 1), jnp.float32)], axis=1)
  parts = _sc_aggregate(x_aug, edge_index.astype(jnp.int32))
  w_comb = jnp.concatenate(
      [W_l.T, b_l[None, :],
       jnp.zeros((D_AUG - D_IN - 1, D_OUT), jnp.float32)], axis=0)
  return _combine(parts, x, w_comb, W_r.T)


# speedup vs baseline: 1.2971x; 1.0581x over previous
import functools

import jax
import jax.numpy as jnp
from jax import lax
from jax.experimental import pallas as pl
from jax.experimental.pallas import tpu as pltpu
from jax.experimental.pallas import tpu_sc as plsc

N_NODES = 10000
N_EDGES = 320000
D_IN = 128
D_OUT = 128

NC = 2
NS = 16
NW = NC * NS
CHUNK = 128
NCHUNKS = N_EDGES // CHUNK
NCT = (NCHUNKS + NW - 1) // NW
N_PAD = 10240
ROWS_PER_TILE = N_PAD // NS
DEG_W = 16


def _sc_aggregate(x, src, dst, zer):
  mesh = plsc.VectorSubcoreMesh(core_axis_name="c", subcore_axis_name="s")

  @functools.partial(
      pl.kernel,
      mesh=mesh,
      out_type=jax.ShapeDtypeStruct((NC, N_PAD, D_IN), jnp.float32),
      scratch_types=[
          pltpu.VMEM((CHUNK,), jnp.int32),
          pltpu.VMEM((CHUNK,), jnp.int32),
          pltpu.VMEM((CHUNK,), jnp.int32),
          pltpu.VMEM((CHUNK,), jnp.int32),
          pltpu.VMEM((CHUNK,), jnp.int32),
          pltpu.VMEM((CHUNK,), jnp.int32),
          pltpu.VMEM((CHUNK,), jnp.int32),
          pltpu.VMEM((CHUNK,), jnp.int32),
          pltpu.VMEM((CHUNK, D_IN), jnp.float32),
          pltpu.VMEM((CHUNK, D_IN), jnp.float32),
          pltpu.VMEM((64, D_IN), jnp.float32),
          pltpu.VMEM_SHARED((N_PAD, D_IN), jnp.float32),
          pltpu.SemaphoreType.DMA,
          pltpu.SemaphoreType.DMA,
          pltpu.SemaphoreType.DMA,
          pltpu.SemaphoreType.DMA,
          pltpu.SemaphoreType.DMA,
          pltpu.SemaphoreType.DMA,
          pltpu.SemaphoreType.DMA,
          pltpu.SemaphoreType.DMA,
      ],
  )
  def body(x_hbm, src_hbm, dst_hbm, zer_hbm, out_hbm, s0, s1, s2, s3,
           d0, d1, d2, d3, rows0, rows1, zbuf, acc,
           is0, is1, is2, is3, gs0, gs1, ss0, ss1):
    c = lax.axis_index("c")
    s = lax.axis_index("s")
    wid = s * NC + c

    sbufs = (s0, s1, s2, s3)
    dbufs = (d0, d1, d2, d3)
    isems = (is0, is1, is2, is3)
    rbufs = (rows0, rows1)
    gsems = (gs0, gs1)
    ssems = (ss0, ss1)

    pltpu.sync_copy(zer_hbm, zbuf)

    def zero_acc(i, carry):
      pltpu.sync_copy(zbuf, acc.at[pl.ds(s * ROWS_PER_TILE + i * 64, 64)])
      return carry

    lax.fori_loop(0, ROWS_PER_TILE // 64, zero_acc, 0)
    plsc.subcore_barrier()

    def cid_of(j):
      return j * NW + wid

    def start_idx(j, ib):
      off = jnp.minimum(cid_of(j), NCHUNKS - 1) * CHUNK
      pltpu.async_copy(src_hbm.at[pl.ds(off, CHUNK)], sbufs[ib], isems[ib])
      pltpu.async_copy(dst_hbm.at[pl.ds(off, CHUNK)], dbufs[ib], isems[ib])

    def wait_idx(ib):
      pltpu.make_async_copy(src_hbm.at[pl.ds(0, CHUNK)], sbufs[ib],
                            isems[ib]).wait()
      pltpu.make_async_copy(dst_hbm.at[pl.ds(0, CHUNK)], dbufs[ib],
                            isems[ib]).wait()

    def start_gather(ib, rb):
      pltpu.async_copy(x_hbm.at[sbufs[ib]], rbufs[rb], gsems[rb])

    def wait_gather(rb):
      pltpu.make_async_copy(x_hbm.at[pl.ds(0, CHUNK)], rbufs[rb],
                            gsems[rb]).wait()

    def start_scatter(ib, rb):
      pltpu.async_copy(rbufs[rb], acc.at[dbufs[ib]], ssems[rb], add=True)

    def wait_scatter(rb):
      pltpu.make_async_copy(rbufs[rb], acc.at[pl.ds(0, CHUNK)],
                            ssems[rb]).wait()

    pltpu.sync_copy(src_hbm.at[pl.ds(wid * CHUNK, CHUNK)], s0)
    pltpu.sync_copy(dst_hbm.at[pl.ds(wid * CHUNK, CHUNK)], d0)
    start_gather(0, 0)
    start_idx(1, 1)

    def quad(p, carry):
      j0 = 4 * p
      for b in range(4):
        j = j0 + b
        ib = b
        rb = b % 2

        @pl.when(j < NCT)
        def _():
          @pl.when(j + 1 < NCT)
          def _():
            wait_idx((ib + 1) % 4)

            @pl.when(j >= 1)
            def _():
              wait_scatter(1 - rb)

            start_gather((ib + 1) % 4, 1 - rb)

          @pl.when(j + 2 < NCT)
          def _():
            start_idx(j + 2, (ib + 2) % 4)

          wait_gather(rb)

          @pl.when(cid_of(j) < NCHUNKS)
          def _():
            start_scatter(ib, rb)

      return carry

    lax.fori_loop(0, (NCT + 3) // 4, quad, 0)
    wait_scatter((NCT - 2) % 2)

    @pl.when(cid_of(NCT - 1) < NCHUNKS)
    def _():
      wait_scatter((NCT - 1) % 2)

    plsc.subcore_barrier()

    def out_step(i, carry):
      r0 = s * ROWS_PER_TILE + i * CHUNK
      pltpu.sync_copy(acc.at[pl.ds(r0, CHUNK)], rows0)
      pltpu.sync_copy(rows0, out_hbm.at[c, pl.ds(r0, CHUNK)])
      return carry

    lax.fori_loop(0, ROWS_PER_TILE // CHUNK, out_step, 0)

  return body(x, src, dst, zer)


def _sc_degree(dst):
  mesh = plsc.VectorSubcoreMesh(core_axis_name="c", subcore_axis_name="s")

  @functools.partial(
      pl.kernel,
      mesh=mesh,
      compiler_params=pltpu.CompilerParams(use_tc_tiling_on_sc=False),
      out_type=jax.ShapeDtypeStruct((NC, N_PAD, DEG_W), jnp.float32),
      scratch_types=[
          pltpu.VMEM((CHUNK,), jnp.int32),
          pltpu.VMEM((CHUNK,), jnp.int32),
          pltpu.VMEM((CHUNK, DEG_W), jnp.float32),
          pltpu.VMEM((CHUNK, DEG_W), jnp.float32),
          pltpu.VMEM_SHARED((N_PAD, DEG_W), jnp.float32),
          pltpu.SemaphoreType.DMA,
          pltpu.SemaphoreType.DMA,
      ],
  )
  def body(dst_hbm, out_hbm, d0, d1, ones, zbuf, dacc, is0, is1):
    c = lax.axis_index("c")
    s = lax.axis_index("s")
    wid = s * NC + c

    dbufs = (d0, d1)
    isems = (is0, is1)

    def fill(r, carry):
      ones[r, pl.ds(0, 16)] = jnp.ones((16,), jnp.float32)
      zbuf[r, pl.ds(0, 16)] = jnp.zeros((16,), jnp.float32)
      return carry

    lax.fori_loop(0, CHUNK, fill, 0)

    def zero_acc(i, carry):
      pltpu.sync_copy(zbuf, dacc.at[pl.ds(s * ROWS_PER_TILE + i * CHUNK,
                                          CHUNK)])
      return carry

    lax.fori_loop(0, ROWS_PER_TILE // CHUNK, zero_acc, 0)
    plsc.subcore_barrier()

    def cid_of(j):
      return j * NW + wid

    def start_idx(j, ib):
      off = jnp.minimum(cid_of(j), NCHUNKS - 1) * CHUNK
      pltpu.async_copy(dst_hbm.at[pl.ds(off, CHUNK)], dbufs[ib], isems[ib])

    def wait_idx(ib):
      pltpu.make_async_copy(dst_hbm.at[pl.ds(0, CHUNK)], dbufs[ib],
                            isems[ib]).wait()

    pltpu.sync_copy(dst_hbm.at[pl.ds(wid * CHUNK, CHUNK)], d0)
    start_idx(1, 1)

    def pair(p, carry):
      j0 = 2 * p
      for b in range(2):
        j = j0 + b
        ib = b

        @pl.when(j < NCT)
        def _():
          @pl.when(j >= 1)
          def _():
            wait_idx(ib)

          @pl.when(cid_of(j) < NCHUNKS)
          def _():
            pltpu.sync_copy(ones, dacc.at[dbufs[ib]], add=True)

          @pl.when(j + 2 < NCT)
          def _():
            start_idx(j + 2, ib)

      return carry

    lax.fori_loop(0, (NCT + 1) // 2, pair, 0)
    plsc.subcore_barrier()

    def out_step(i, carry):
      r0 = s * ROWS_PER_TILE + i * CHUNK
      pltpu.sync_copy(dacc.at[pl.ds(r0, CHUNK)], zbuf)
      pltpu.sync_copy(zbuf, out_hbm.at[c, pl.ds(r0, CHUNK)])
      return carry

    lax.fori_loop(0, ROWS_PER_TILE // CHUNK, out_step, 0)

  return body(dst)


BLK = 1000


def _combine(parts, degp, x, w_l_t, b_l_row, w_r_t):
  def body(p_ref, dp_ref, x_ref, wl_ref, bl_ref, wr_ref, o_ref):
    acc = p_ref[0] + p_ref[1]
    deg = dp_ref[0, :, 0:1] + dp_ref[1, :, 0:1]
    o_ref[...] = (
        jnp.dot(acc, wl_ref[...], preferred_element_type=jnp.float32,
                precision=lax.Precision.HIGHEST)
        + deg * bl_ref[...]
        + jnp.dot(x_ref[...], wr_ref[...], preferred_element_type=jnp.float32,
                  precision=lax.Precision.HIGHEST))

  return pl.pallas_call(
      body,
      grid=(N_NODES // BLK,),
      in_specs=[
          pl.BlockSpec((NC, BLK, D_IN), lambda i: (0, i, 0)),
          pl.BlockSpec((NC, BLK, DEG_W), lambda i: (0, i, 0)),
          pl.BlockSpec((BLK, D_IN), lambda i: (i, 0)),
          pl.BlockSpec((D_IN, D_OUT), lambda i: (0, 0)),
          pl.BlockSpec((1, D_OUT), lambda i: (0, 0)),
          pl.BlockSpec((D_IN, D_OUT), lambda i: (0, 0)),
      ],
      out_specs=pl.BlockSpec((BLK, D_OUT), lambda i: (i, 0)),
      out_shape=jax.ShapeDtypeStruct((N_NODES, D_OUT), jnp.float32),
  )(parts, degp, x, w_l_t, b_l_row, w_r_t)


def kernel(x, edge_index, W_l, b_l, W_r):
  src = edge_index[0].astype(jnp.int32)
  dst = edge_index[1].astype(jnp.int32)
  zer = jnp.zeros((64, D_IN), jnp.float32)
  parts = _sc_aggregate(x, src, dst, zer)
  degp = _sc_degree(dst)
  return _combine(parts, degp, x, W_l.T, b_l[None, :], W_r.T)


# speedup vs baseline: 1.4060x; 1.0840x over previous
import functools

import jax
import jax.numpy as jnp
from jax import lax
from jax.experimental import pallas as pl
from jax.experimental.pallas import tpu as pltpu
from jax.experimental.pallas import tpu_sc as plsc

N_NODES = 10000
N_EDGES = 320000
D_IN = 128
D_OUT = 128

NC = 2
NS = 16
NW = NC * NS
CHUNK = 128
NCHUNKS = N_EDGES // CHUNK
NCT = (NCHUNKS + NW - 1) // NW
N_PAD = 10240
ROWS_PER_TILE = N_PAD // NS
DEG_W = 16


def _sc_aggregate(x, eflat, zer):
  mesh = plsc.VectorSubcoreMesh(core_axis_name="c", subcore_axis_name="s")

  @functools.partial(
      pl.kernel,
      mesh=mesh,
      out_type=jax.ShapeDtypeStruct((NC, N_PAD, D_IN), jnp.float32),
      scratch_types=[
          pltpu.VMEM((CHUNK,), jnp.int32),
          pltpu.VMEM((CHUNK,), jnp.int32),
          pltpu.VMEM((CHUNK,), jnp.int32),
          pltpu.VMEM((CHUNK,), jnp.int32),
          pltpu.VMEM((CHUNK,), jnp.int32),
          pltpu.VMEM((CHUNK,), jnp.int32),
          pltpu.VMEM((CHUNK,), jnp.int32),
          pltpu.VMEM((CHUNK,), jnp.int32),
          pltpu.VMEM((CHUNK, D_IN), jnp.float32),
          pltpu.VMEM((CHUNK, D_IN), jnp.float32),
          pltpu.VMEM((64, D_IN), jnp.float32),
          pltpu.VMEM_SHARED((N_PAD, D_IN), jnp.float32),
          pltpu.SemaphoreType.DMA,
          pltpu.SemaphoreType.DMA,
          pltpu.SemaphoreType.DMA,
          pltpu.SemaphoreType.DMA,
          pltpu.SemaphoreType.DMA,
          pltpu.SemaphoreType.DMA,
          pltpu.SemaphoreType.DMA,
          pltpu.SemaphoreType.DMA,
      ],
  )
  def body(x_hbm, e_hbm, zer_hbm, out_hbm, s0, s1, s2, s3,
           d0, d1, d2, d3, rows0, rows1, zbuf, acc,
           is0, is1, is2, is3, gs0, gs1, ss0, ss1):
    c = lax.axis_index("c")
    s = lax.axis_index("s")
    wid = s * NC + c

    sbufs = (s0, s1, s2, s3)
    dbufs = (d0, d1, d2, d3)
    isems = (is0, is1, is2, is3)
    rbufs = (rows0, rows1)
    gsems = (gs0, gs1)
    ssems = (ss0, ss1)

    pltpu.sync_copy(zer_hbm, zbuf)

    def zero_acc(i, carry):
      pltpu.sync_copy(zbuf, acc.at[pl.ds(s * ROWS_PER_TILE + i * 64, 64)])
      return carry

    lax.fori_loop(0, ROWS_PER_TILE // 64, zero_acc, 0)
    plsc.subcore_barrier()

    def cid_of(j):
      return j * NW + wid

    def start_idx(j, ib):
      off = jnp.minimum(cid_of(j), NCHUNKS - 1) * CHUNK
      pltpu.async_copy(e_hbm.at[pl.ds(off, CHUNK)], sbufs[ib], isems[ib])
      pltpu.async_copy(e_hbm.at[pl.ds(N_EDGES + off, CHUNK)], dbufs[ib],
                       isems[ib])

    def wait_idx(ib):
      pltpu.make_async_copy(e_hbm.at[pl.ds(0, CHUNK)], sbufs[ib],
                            isems[ib]).wait()
      pltpu.make_async_copy(e_hbm.at[pl.ds(0, CHUNK)], dbufs[ib],
                            isems[ib]).wait()

    def start_gather(ib, rb):
      pltpu.async_copy(x_hbm.at[sbufs[ib]], rbufs[rb], gsems[rb])

    def wait_gather(rb):
      pltpu.make_async_copy(x_hbm.at[pl.ds(0, CHUNK)], rbufs[rb],
                            gsems[rb]).wait()

    def start_scatter(ib, rb):
      pltpu.async_copy(rbufs[rb], acc.at[dbufs[ib]], ssems[rb], add=True)

    def wait_scatter(rb):
      pltpu.make_async_copy(rbufs[rb], acc.at[pl.ds(0, CHUNK)],
                            ssems[rb]).wait()

    pltpu.sync_copy(e_hbm.at[pl.ds(wid * CHUNK, CHUNK)], s0)
    pltpu.sync_copy(e_hbm.at[pl.ds(N_EDGES + wid * CHUNK, CHUNK)], d0)
    start_gather(0, 0)
    start_idx(1, 1)

    def quad(p, carry):
      j0 = 4 * p
      for b in range(4):
        j = j0 + b
        ib = b
        rb = b % 2

        @pl.when(j < NCT)
        def _():
          @pl.when(j + 1 < NCT)
          def _():
            wait_idx((ib + 1) % 4)

            @pl.when(j >= 1)
            def _():
              wait_scatter(1 - rb)

            start_gather((ib + 1) % 4, 1 - rb)

          @pl.when(j + 2 < NCT)
          def _():
            start_idx(j + 2, (ib + 2) % 4)

          wait_gather(rb)

          @pl.when(cid_of(j) < NCHUNKS)
          def _():
            start_scatter(ib, rb)

      return carry

    lax.fori_loop(0, (NCT + 3) // 4, quad, 0)
    wait_scatter((NCT - 2) % 2)

    @pl.when(cid_of(NCT - 1) < NCHUNKS)
    def _():
      wait_scatter((NCT - 1) % 2)

    plsc.subcore_barrier()

    def out_step(i, carry):
      r0 = s * ROWS_PER_TILE + i * CHUNK
      pltpu.sync_copy(acc.at[pl.ds(r0, CHUNK)], rows0)
      pltpu.sync_copy(rows0, out_hbm.at[c, pl.ds(r0, CHUNK)])
      return carry

    lax.fori_loop(0, ROWS_PER_TILE // CHUNK, out_step, 0)

  return body(x, eflat, zer)


def _sc_degree(eflat):
  mesh = plsc.VectorSubcoreMesh(core_axis_name="c", subcore_axis_name="s")

  @functools.partial(
      pl.kernel,
      mesh=mesh,
      compiler_params=pltpu.CompilerParams(use_tc_tiling_on_sc=False),
      out_type=jax.ShapeDtypeStruct((NC, N_PAD, DEG_W), jnp.float32),
      scratch_types=[
          pltpu.VMEM((CHUNK,), jnp.int32),
          pltpu.VMEM((CHUNK,), jnp.int32),
          pltpu.VMEM((CHUNK,), jnp.int32),
          pltpu.VMEM((CHUNK,), jnp.int32),
          pltpu.VMEM((CHUNK, DEG_W), jnp.float32),
          pltpu.VMEM((CHUNK, DEG_W), jnp.float32),
          pltpu.VMEM_SHARED((N_PAD, DEG_W), jnp.float32),
          pltpu.SemaphoreType.DMA,
          pltpu.SemaphoreType.DMA,
          pltpu.SemaphoreType.DMA,
          pltpu.SemaphoreType.DMA,
          pltpu.SemaphoreType.DMA,
          pltpu.SemaphoreType.DMA,
          pltpu.SemaphoreType.DMA,
          pltpu.SemaphoreType.DMA,
      ],
  )
  def body(e_hbm, out_hbm, d0, d1, d2, d3, ones, zbuf, dacc,
           is0, is1, is2, is3, ss0, ss1, ss2, ss3):
    c = lax.axis_index("c")
    s = lax.axis_index("s")
    wid = s * NC + c

    dbufs = (d0, d1, d2, d3)
    isems = (is0, is1, is2, is3)
    ssems = (ss0, ss1, ss2, ss3)

    def fill(r, carry):
      ones[r, pl.ds(0, 16)] = jnp.ones((16,), jnp.float32)
      zbuf[r, pl.ds(0, 16)] = jnp.zeros((16,), jnp.float32)
      return carry

    lax.fori_loop(0, CHUNK, fill, 0)

    def zero_acc(i, carry):
      pltpu.sync_copy(zbuf, dacc.at[pl.ds(s * ROWS_PER_TILE + i * CHUNK,
                                          CHUNK)])
      return carry

    lax.fori_loop(0, ROWS_PER_TILE // CHUNK, zero_acc, 0)
    plsc.subcore_barrier()

    def cid_of(j):
      return j * NW + wid

    def start_idx(j, ib):
      off = jnp.minimum(cid_of(j), NCHUNKS - 1) * CHUNK
      pltpu.async_copy(e_hbm.at[pl.ds(N_EDGES + off, CHUNK)], dbufs[ib],
                       isems[ib])

    def wait_idx(ib):
      pltpu.make_async_copy(e_hbm.at[pl.ds(0, CHUNK)], dbufs[ib],
                            isems[ib]).wait()

    def wait_scatter(k):
      pltpu.make_async_copy(ones, dacc.at[pl.ds(0, CHUNK)], ssems[k]).wait()

    pltpu.sync_copy(e_hbm.at[pl.ds(N_EDGES + wid * CHUNK, CHUNK)], d0)
    start_idx(1, 1)

    def quad(p, carry):
      j0 = 4 * p
      for b in range(4):
        j = j0 + b
        ib = b

        @pl.when(j < NCT)
        def _():
          @pl.when(j >= 1)
          def _():
            wait_idx(ib)

          @pl.when(cid_of(j) < NCHUNKS)
          def _():
            pltpu.async_copy(ones, dacc.at[dbufs[ib]], ssems[ib], add=True)

          @pl.when(j + 2 < NCT)
          def _():
            @pl.when(j >= 2)
            def _():
              wait_scatter((ib + 2) % 4)

            start_idx(j + 2, (ib + 2) % 4)

      return carry

    lax.fori_loop(0, (NCT + 3) // 4, quad, 0)
    wait_scatter((NCT - 4) % 4)
    wait_scatter((NCT - 3) % 4)
    wait_scatter((NCT - 2) % 4)

    @pl.when(cid_of(NCT - 1) < NCHUNKS)
    def _():
      wait_scatter((NCT - 1) % 4)

    plsc.subcore_barrier()

    def out_step(i, carry):
      r0 = s * ROWS_PER_TILE + i * CHUNK
      pltpu.sync_copy(dacc.at[pl.ds(r0, CHUNK)], zbuf)
      pltpu.sync_copy(zbuf, out_hbm.at[c, pl.ds(r0, CHUNK)])
      return carry

    lax.fori_loop(0, ROWS_PER_TILE // CHUNK, out_step, 0)

  return body(eflat)


BLK = 1000


def _combine(parts, degp, x, w_l_t, b_l_row, w_r_t):
  def body(p_ref, dp_ref, x_ref, wl_ref, bl_ref, wr_ref, o_ref):
    acc = p_ref[0] + p_ref[1]
    deg = dp_ref[0, :, 0:1] + dp_ref[1, :, 0:1]
    o_ref[...] = (
        jnp.dot(acc, wl_ref[...], preferred_element_type=jnp.float32,
                precision=lax.Precision.HIGHEST)
        + deg * bl_ref[...]
        + jnp.dot(x_ref[...], wr_ref[...], preferred_element_type=jnp.float32,
                  precision=lax.Precision.HIGHEST))

  return pl.pallas_call(
      body,
      grid=(N_NODES // BLK,),
      in_specs=[
          pl.BlockSpec((NC, BLK, D_IN), lambda i: (0, i, 0)),
          pl.BlockSpec((NC, BLK, DEG_W), lambda i: (0, i, 0)),
          pl.BlockSpec((BLK, D_IN), lambda i: (i, 0)),
          pl.BlockSpec((D_IN, D_OUT), lambda i: (0, 0)),
          pl.BlockSpec((1, D_OUT), lambda i: (0, 0)),
          pl.BlockSpec((D_IN, D_OUT), lambda i: (0, 0)),
      ],
      out_specs=pl.BlockSpec((BLK, D_OUT), lambda i: (i, 0)),
      out_shape=jax.ShapeDtypeStruct((N_NODES, D_OUT), jnp.float32),
  )(parts, degp, x, w_l_t, b_l_row, w_r_t)


def kernel(x, edge_index, W_l, b_l, W_r):
  eflat = edge_index.astype(jnp.int32).reshape(2 * N_EDGES)
  zer = jnp.zeros((64, D_IN), jnp.float32)
  parts = _sc_aggregate(x, eflat, zer)
  degp = _sc_degree(eflat)
  return _combine(parts, degp, x, W_l.T, b_l[None, :], W_r.T)
